# baseline (speedup 1.0000x reference)
import functools

import jax
import jax.numpy as jnp
from jax import lax
from jax.experimental import pallas as pl
from jax.experimental.pallas import tpu as pltpu
from jax.experimental.pallas import tpu_sc as plsc

N = 10000
E = 160000
GAMMA = 0.1
RB = 400
NRB = N // RB

C = 128
NTILE = 16
FH = 128
E_PAD = 4096 * ((E + 4095) // 4096)
RPT = 624
TAIL = N - NTILE * RPT


def _make_spmm(edge_split):
    per_tile = E_PAD // (32 if edge_split else 16)
    n_chunks = per_tile // C
    n_groups = n_chunks // 4
    mesh = plsc.VectorSubcoreMesh(core_axis_name="c", subcore_axis_name="s",
                                  num_cores=2, num_subcores=NTILE)

    @functools.partial(
        pl.kernel,
        out_type=jax.ShapeDtypeStruct((2 * N, FH), jnp.float32),
        mesh=mesh,
        scratch_types=[
            pltpu.VMEM((4, 2, C), jnp.int32),
            pltpu.VMEM((4 * C * 16,), jnp.float32),
            pltpu.VMEM((2, C, FH), jnp.float32),
            pltpu.VMEM_SHARED((N, FH), jnp.float32),
        ] + [pltpu.SemaphoreType.DMA] * 12,
    )
    def spmm(table_hbm, pk_hbm, vals_hbm, zeros_hbm, out_hbm,
             pk_v, vals_v, rows, acc,
             p0, p1, p2, p3, v0, v1, v2, v3, g0, g1, s0, s1):
        psem = [p0, p1, p2, p3]
        vsem = [v0, v1, v2, v3]
        gsem = [g0, g1]
        ssem = [s0, s1]
        c = lax.axis_index("c")
        s = lax.axis_index("s")
        base = s * per_tile
        if edge_split:
            base = base + c * (E_PAD // 2)

        VCH = C * 16

        def issue_pk(j, bq):
            off = pl.multiple_of(base + j * C, C)
            pltpu.async_copy(pk_hbm.at[c, :, pl.ds(off, C)], pk_v.at[bq], psem[bq])
            pltpu.async_copy(vals_hbm.at[pl.ds(off * 16, VCH)],
                             vals_v.at[pl.ds(bq * VCH, VCH)], vsem[bq])

        def drain_pk(bq):
            pltpu.make_async_copy(pk_hbm.at[c, :, pl.ds(0, C)],
                                  pk_v.at[bq], psem[bq]).wait()

        def drain_vals(bq):
            pltpu.make_async_copy(vals_hbm.at[pl.ds(0, VCH)],
                                  vals_v.at[pl.ds(bq * VCH, VCH)], vsem[bq]).wait()

        def issue_gather(bq, br):
            pltpu.async_copy(table_hbm.at[pk_v.at[bq, 0]], rows.at[br], gsem[br])

        def drain_gather(br):
            pltpu.make_async_copy(table_hbm.at[pl.ds(0, C)],
                                  rows.at[br], gsem[br]).wait()

        def issue_scatter(bq, br):
            pltpu.async_copy(rows.at[br], acc.at[pk_v.at[bq, 1]], ssem[br],
                             add=True)

        def drain_scatter(br):
            pltpu.make_async_copy(table_hbm.at[pl.ds(0, C)],
                                  rows.at[br], ssem[br]).wait()

        pltpu.sync_copy(zeros_hbm, acc.at[pl.ds(s * RPT, RPT)])

        @pl.when(s == NTILE - 1)
        def _zero_tail():
            pltpu.sync_copy(zeros_hbm.at[pl.ds(0, TAIL)],
                            acc.at[pl.ds(NTILE * RPT, TAIL)])

        plsc.subcore_barrier()

        issue_pk(0, 0)
        issue_pk(1, 1)
        drain_pk(0)
        issue_gather(0, 0)

        def group(gi, cy):
            for u in range(4):
                br = u % 2
                bq = u
                j = 4 * gi + u
                drain_gather(br)

                @pl.when(j + 1 < n_chunks)
                def _prep(j=j, br=br, bq=bq):
                    drain_pk((bq + 1) % 4)

                    @pl.when(j >= 1)
                    def _():
                        drain_scatter(1 - br)

                    issue_gather((bq + 1) % 4, 1 - br)

                drain_vals(bq)

                def scale(e, cy2, br=br, bq=bq):
                    vs = vals_v[pl.ds(bq * VCH + e * 16, 16)]
                    for f in range(FH // 16):
                        sl = pl.ds(f * 16, 16)
                        rows[br, e, sl] = rows[br, e, sl] * vs
                    return cy2

                lax.fori_loop(0, C, scale, 0)
                issue_scatter(bq, br)

                @pl.when(j + 2 < n_chunks)
                def _next(j=j, bq=bq):
                    issue_pk(j + 2, (bq + 2) % 4)

            return cy

        lax.fori_loop(0, n_groups, group, 0)
        drain_scatter(0)
        drain_scatter(1)
        plsc.subcore_barrier()
        pltpu.sync_copy(acc.at[pl.ds(s * RPT, RPT)],
                        out_hbm.at[pl.ds(c * N + s * RPT, RPT)])

        @pl.when(s == NTILE - 1)
        def _flush_tail():
            pltpu.sync_copy(acc.at[pl.ds(NTILE * RPT, TAIL)],
                            out_hbm.at[pl.ds(c * N + NTILE * RPT, TAIL)])

    return spmm


@functools.cache
def _get_spmm(edge_split):
    return _make_spmm(edge_split)


def _prep_edges(ei, vals):
    pad = E_PAD - E
    src = jnp.concatenate([ei[1], jnp.zeros((pad,), jnp.int32)])
    dst = jnp.concatenate([ei[0], jnp.zeros((pad,), jnp.int32)])
    v = jnp.concatenate([vals, jnp.zeros((pad,), jnp.float32)])
    vx = jnp.broadcast_to(v[:, None], (E_PAD, 16)).reshape(-1)
    pkc = jnp.stack([jnp.stack([src, dst]), jnp.stack([src + N, dst])])
    pke = jnp.stack([jnp.stack([src, dst])] * 2)
    return pkc, pke, vx


_row = lambda i: (i, 0)
_rowlo = lambda i: (i, 0)
_rowhi = lambda i: (i + NRB, 0)
_full2 = lambda i: (0, 0)
_full1 = lambda i: (0,)
_s3 = lambda i: (0, i, 0)


def _bs(shape, imap):
    return pl.BlockSpec(shape, imap)


def _layer_in_body(f_ref, w_ref, ws_ref, sc_ref, dk_ref, b0_ref, d0_ref,
                   h2_ref, hs_ref, si_ref, dki_ref):
    f = f_ref[...]
    h = jnp.dot(f, w_ref[...], preferred_element_type=jnp.float32)
    hs = jnp.dot(f, ws_ref[...], preferred_element_type=jnp.float32)
    si_ref[...] = jax.nn.sigmoid(
        jnp.dot(f, sc_ref[...], preferred_element_type=jnp.float32) + b0_ref[0, 0])
    dki_ref[...] = jnp.dot(f, dk_ref[...],
                           preferred_element_type=jnp.float32) + d0_ref[0, 0]
    h2_ref[0] = h[:, :FH]
    h2_ref[1] = h[:, FH:]
    hs_ref[...] = hs


def _layer_in(fea, W, Ws, scores0, Dk0, bias0, Dbias0):
    out = pl.pallas_call(
        _layer_in_body,
        grid=(NRB,),
        in_specs=[
            _bs((RB, 256), _row), _bs((256, 256), _full2), _bs((256, 256), _full2),
            _bs((256, 1), _full2), _bs((256, 1), _full2),
            _bs((1, 1), _full2), _bs((1, 1), _full2),
        ],
        out_specs=[
            _bs((2, RB, FH), _s3), _bs((RB, 256), _row),
            _bs((RB, 1), _row), _bs((RB, 1), _row),
        ],
        out_shape=[
            jax.ShapeDtypeStruct((2, N, FH), jnp.float32),
            jax.ShapeDtypeStruct((N, 256), jnp.float32),
            jax.ShapeDtypeStruct((N, 1), jnp.float32),
            jax.ShapeDtypeStruct((N, 1), jnp.float32),
        ],
    )(fea, W, Ws, scores0, Dk0, bias0.reshape(1, 1), Dbias0.reshape(1, 1))
    h2, hs, si, dki = out
    return h2.reshape(2 * N, FH), hs, si, dki


def _layer_mid_body(aa_ref, ab_ref, ka_ref, kb_ref, ha_ref, hb_ref, hs_ref,
                    si_ref, dki_ref, b_ref, wm_ref, wsm_ref, bm_ref,
                    ym2_ref, zm_ref):
    a = jnp.concatenate([aa_ref[...], ab_ref[...]], axis=1)
    k = jnp.concatenate([ka_ref[...], kb_ref[...]], axis=1)
    h = jnp.concatenate([ha_ref[...], hb_ref[...]], axis=1)
    hs = hs_ref[...]
    si = si_ref[...]
    dki = dki_ref[...]
    b = b_ref[...][None, :]
    x1 = si * a + (1.0 - si) * k + hs + b + GAMMA * dki * (h + hs + b)
    ym = jnp.dot(x1, wm_ref[...], preferred_element_type=jnp.float32)
    zm = jnp.dot(x1, wsm_ref[...], preferred_element_type=jnp.float32)
    ym2_ref[0] = ym[:, :FH]
    ym2_ref[1] = ym[:, FH:]
    zm_ref[...] = zm + bm_ref[...][None, :]


def _layer_mid(a2, k2, h2, hs, si, dki, b_in, W_mid, Ws_mid, b_mid):
    out = pl.pallas_call(
        _layer_mid_body,
        grid=(NRB,),
        in_specs=[
            _bs((RB, FH), _rowlo), _bs((RB, FH), _rowhi),
            _bs((RB, FH), _rowlo), _bs((RB, FH), _rowhi),
            _bs((RB, FH), _rowlo), _bs((RB, FH), _rowhi),
            _bs((RB, 256), _row), _bs((RB, 1), _row), _bs((RB, 1), _row),
            _bs((256,), _full1),
            _bs((256, 256), _full2), _bs((256, 256), _full2), _bs((256,), _full1),
        ],
        out_specs=[_bs((2, RB, FH), _s3), _bs((RB, 256), _row)],
        out_shape=[
            jax.ShapeDtypeStruct((2, N, FH), jnp.float32),
            jax.ShapeDtypeStruct((N, 256), jnp.float32),
        ],
    )(a2, a2, k2, k2, h2, h2, hs, si, dki, b_in, W_mid, Ws_mid, b_mid)
    ym2, zm = out
    return ym2.reshape(2 * N, FH), zm


def _layer_out_body(ma_ref, mb_ref, zm_ref, sc_ref, dk_ref, b1_ref, d1_ref,
                    wo_ref, wso_ref, g128_ref, gs_ref, so_ref, dko_ref):
    x2 = jnp.concatenate([ma_ref[...], mb_ref[...]], axis=1) + zm_ref[...]
    so_ref[...] = jax.nn.sigmoid(
        jnp.dot(x2, sc_ref[...], preferred_element_type=jnp.float32) + b1_ref[0, 0])
    dko_ref[...] = jnp.dot(x2, dk_ref[...],
                           preferred_element_type=jnp.float32) + d1_ref[0, 0]
    g = jnp.dot(x2, wo_ref[...], preferred_element_type=jnp.float32)
    gs_ref[...] = jnp.dot(x2, wso_ref[...], preferred_element_type=jnp.float32)
    g128_ref[...] = jnp.pad(g, ((0, 0), (0, FH - NCLASS)))


NCLASS = 64


def _layer_out(m2, zm, scores1, Dk1, bias1, Dbias1, W_out, Ws_out):
    return pl.pallas_call(
        _layer_out_body,
        grid=(NRB,),
        in_specs=[
            _bs((RB, FH), _rowlo), _bs((RB, FH), _rowhi),
            _bs((RB, 256), _row),
            _bs((256, 1), _full2), _bs((256, 1), _full2),
            _bs((1, 1), _full2), _bs((1, 1), _full2),
            _bs((256, NCLASS), _full2), _bs((256, NCLASS), _full2),
        ],
        out_specs=[
            _bs((RB, FH), _row), _bs((RB, NCLASS), _row),
            _bs((RB, 1), _row), _bs((RB, 1), _row),
        ],
        out_shape=[
            jax.ShapeDtypeStruct((N, FH), jnp.float32),
            jax.ShapeDtypeStruct((N, NCLASS), jnp.float32),
            jax.ShapeDtypeStruct((N, 1), jnp.float32),
            jax.ShapeDtypeStruct((N, 1), jnp.float32),
        ],
    )(m2, m2, zm, scores1, Dk1, bias1.reshape(1, 1), Dbias1.reshape(1, 1),
      W_out, Ws_out)


def _outcomb_body(oaa_ref, oab_ref, oka_ref, okb_ref, g128_ref, gs_ref,
                  s_ref, dk_ref, b_ref, o_ref):
    oa = oaa_ref[...][:, :NCLASS] + oab_ref[...][:, :NCLASS]
    ok = oka_ref[...][:, :NCLASS] + okb_ref[...][:, :NCLASS]
    g = g128_ref[...][:, :NCLASS]
    s = s_ref[...]
    dk = dk_ref[...]
    b = b_ref[...][None, :]
    gs = gs_ref[...]
    idt = g + gs + b
    x = s * oa + (1.0 - s) * ok + gs + b + GAMMA * dk * idt
    m = jnp.max(x, axis=1, keepdims=True)
    lse = jnp.log(jnp.sum(jnp.exp(x - m), axis=1, keepdims=True)) + m
    o_ref[...] = x - lse


def _outcomb(oa2, ok2, g128, gs, s, dk, b):
    return pl.pallas_call(
        _outcomb_body,
        grid=(NRB,),
        in_specs=[
            _bs((RB, FH), _rowlo), _bs((RB, FH), _rowhi),
            _bs((RB, FH), _rowlo), _bs((RB, FH), _rowhi),
            _bs((RB, FH), _row), _bs((RB, NCLASS), _row),
            _bs((RB, 1), _row), _bs((RB, 1), _row),
            _bs((NCLASS,), _full1),
        ],
        out_specs=_bs((RB, NCLASS), _row),
        out_shape=jax.ShapeDtypeStruct((N, NCLASS), jnp.float32),
    )(oa2, oa2, ok2, ok2, g128, gs, s, dk, b)


def kernel(fea, adj_edge_index, adj_values, adj_knn_edge_index, adj_knn_values,
           W_in, Ws_in, b_in, W_mid, Ws_mid, b_mid, W_out, Ws_out, b_out,
           scores0, bias0, scores1, bias1, Dk0, Dbias0, Dk1, Dbias1):
    ea = _prep_edges(adj_edge_index, adj_values)
    ek = _prep_edges(adj_knn_edge_index, adj_knn_values)
    zh = jnp.zeros((RPT, FH), jnp.float32)

    spmm_h = _get_spmm(False)
    spmm_o = _get_spmm(True)
    h2, hs, si, dki = _layer_in(fea, W_in, Ws_in, scores0, Dk0, bias0, Dbias0)
    a2 = spmm_h(h2, ea[0], ea[2], zh)
    k2 = spmm_h(h2, ek[0], ek[2], zh)
    ym2, zm = _layer_mid(a2, k2, h2, hs, si, dki, b_in, W_mid, Ws_mid, b_mid)
    m2 = spmm_h(ym2, ea[0], ea[2], zh)
    g128, gs, so, dko = _layer_out(m2, zm, scores1, Dk1, bias1, Dbias1,
                                   W_out, Ws_out)
    oa2 = spmm_o(g128, ea[1], ea[2], zh)
    ok2 = spmm_o(g128, ek[1], ek[2], zh)
    return _outcomb(oa2, ok2, g128, gs, so, dko, b_out)


# speedup vs baseline: 1.0061x; 1.0061x over previous
import functools

import jax
import jax.numpy as jnp
from jax import lax
from jax.experimental import pallas as pl
from jax.experimental.pallas import tpu as pltpu
from jax.experimental.pallas import tpu_sc as plsc

N = 10000
E = 160000
GAMMA = 0.1
RB = 400
NRB = N // RB

C = 128
NTILE = 16
FH = 128
E_PAD = 4096 * ((E + 4095) // 4096)
RPT = 624
TAIL = N - NTILE * RPT


def _make_spmm(edge_split):
    per_tile = E_PAD // (32 if edge_split else 16)
    n_chunks = per_tile // C
    n_groups = n_chunks // 8
    mesh = plsc.VectorSubcoreMesh(core_axis_name="c", subcore_axis_name="s",
                                  num_cores=2, num_subcores=NTILE)

    @functools.partial(
        pl.kernel,
        out_type=jax.ShapeDtypeStruct((2 * N, FH), jnp.float32),
        mesh=mesh,
        scratch_types=[
            pltpu.VMEM((8, 2, C), jnp.int32),
            pltpu.VMEM((4 * C * 16,), jnp.float32),
            pltpu.VMEM((2, C, FH), jnp.float32),
            pltpu.VMEM_SHARED((N, FH), jnp.float32),
        ] + [pltpu.SemaphoreType.DMA] * 16,
    )
    def spmm(table_hbm, pk_hbm, vals_hbm, zeros_hbm, out_hbm,
             pk_v, vals_v, rows, acc, *sems):
        psem = sems[0:8]
        vsem = sems[8:12]
        gsem = sems[12:14]
        ssem = sems[14:16]
        c = lax.axis_index("c")
        s = lax.axis_index("s")
        base = s * per_tile
        if edge_split:
            base = base + c * (E_PAD // 2)

        VCH = C * 16

        def issue_pk(j, bq):
            off = pl.multiple_of(base + j * C, C)
            pltpu.async_copy(pk_hbm.at[c, :, pl.ds(off, C)], pk_v.at[bq], psem[bq])
            bv = bq % 4
            pltpu.async_copy(vals_hbm.at[pl.ds(off * 16, VCH)],
                             vals_v.at[pl.ds(bv * VCH, VCH)], vsem[bv])

        def drain_pk(bq):
            pltpu.make_async_copy(pk_hbm.at[c, :, pl.ds(0, C)],
                                  pk_v.at[bq], psem[bq]).wait()

        def drain_vals(bv):
            pltpu.make_async_copy(vals_hbm.at[pl.ds(0, VCH)],
                                  vals_v.at[pl.ds(bv * VCH, VCH)], vsem[bv]).wait()

        def issue_gather(bq, br):
            pltpu.async_copy(table_hbm.at[pk_v.at[bq, 0]], rows.at[br], gsem[br])

        def drain_gather(br):
            pltpu.make_async_copy(table_hbm.at[pl.ds(0, C)],
                                  rows.at[br], gsem[br]).wait()

        def issue_scatter(bq, br):
            pltpu.async_copy(rows.at[br], acc.at[pk_v.at[bq, 1]], ssem[br],
                             add=True)

        def drain_scatter(br):
            pltpu.make_async_copy(table_hbm.at[pl.ds(0, C)],
                                  rows.at[br], ssem[br]).wait()

        for q in range(4):
            issue_pk(q, q)
        drain_pk(0)
        issue_gather(0, 0)

        pltpu.sync_copy(zeros_hbm, acc.at[pl.ds(s * RPT, RPT)])

        @pl.when(s == NTILE - 1)
        def _zero_tail():
            pltpu.sync_copy(zeros_hbm.at[pl.ds(0, TAIL)],
                            acc.at[pl.ds(NTILE * RPT, TAIL)])

        plsc.subcore_barrier()

        def group(gi, cy):
            for u in range(8):
                br = u % 2
                bv = u % 4
                bq = u
                j = 8 * gi + u
                drain_gather(br)

                @pl.when(j >= 1)
                def _free(j=j, br=br):
                    drain_scatter(1 - br)

                @pl.when(j + 1 < n_chunks)
                def _prep(j=j, br=br, bq=bq):
                    drain_pk((bq + 1) % 8)
                    issue_gather((bq + 1) % 8, 1 - br)

                drain_vals(bv)

                def scale(i, cy2, br=br, bv=bv):
                    for e2 in range(2):
                        e = i * 2 + e2
                        vs = vals_v[pl.ds(bv * VCH + e * 16, 16)]
                        for f in range(FH // 16):
                            sl = pl.ds(f * 16, 16)
                            rows[br, e, sl] = rows[br, e, sl] * vs
                    return cy2

                lax.fori_loop(0, C // 2, scale, 0)
                issue_scatter(bq, br)

                @pl.when(j + 4 < n_chunks)
                def _next(j=j, bq=bq):
                    issue_pk(j + 4, (bq + 4) % 8)

            return cy

        lax.fori_loop(0, n_groups, group, 0)
        drain_scatter((n_chunks - 1) % 2)
        plsc.subcore_barrier()
        pltpu.sync_copy(acc.at[pl.ds(s * RPT, RPT)],
                        out_hbm.at[pl.ds(c * N + s * RPT, RPT)])

        @pl.when(s == NTILE - 1)
        def _flush_tail():
            pltpu.sync_copy(acc.at[pl.ds(NTILE * RPT, TAIL)],
                            out_hbm.at[pl.ds(c * N + NTILE * RPT, TAIL)])

    return spmm


@functools.cache
def _get_spmm(edge_split):
    return _make_spmm(edge_split)


def _prep_edges(ei, vals):
    pad = E_PAD - E
    src = jnp.concatenate([ei[1], jnp.zeros((pad,), jnp.int32)])
    dst = jnp.concatenate([ei[0], jnp.zeros((pad,), jnp.int32)])
    v = jnp.concatenate([vals, jnp.zeros((pad,), jnp.float32)])
    vx = jnp.broadcast_to(v[:, None], (E_PAD, 16)).reshape(-1)
    pkc = jnp.stack([jnp.stack([src, dst]), jnp.stack([src + N, dst])])
    pke = jnp.stack([jnp.stack([src, dst])] * 2)
    return pkc, pke, vx


_row = lambda i: (i, 0)
_rowlo = lambda i: (i, 0)
_rowhi = lambda i: (i + NRB, 0)
_full2 = lambda i: (0, 0)
_full1 = lambda i: (0,)
_s3 = lambda i: (0, i, 0)


def _bs(shape, imap):
    return pl.BlockSpec(shape, imap)


def _layer_in_body(f_ref, w_ref, ws_ref, sc_ref, dk_ref, b0_ref, d0_ref,
                   h2_ref, hs_ref, si_ref, dki_ref):
    f = f_ref[...]
    h = jnp.dot(f, w_ref[...], preferred_element_type=jnp.float32)
    hs = jnp.dot(f, ws_ref[...], preferred_element_type=jnp.float32)
    si_ref[...] = jax.nn.sigmoid(
        jnp.dot(f, sc_ref[...], preferred_element_type=jnp.float32) + b0_ref[0, 0])
    dki_ref[...] = jnp.dot(f, dk_ref[...],
                           preferred_element_type=jnp.float32) + d0_ref[0, 0]
    h2_ref[0] = h[:, :FH]
    h2_ref[1] = h[:, FH:]
    hs_ref[...] = hs


def _layer_in(fea, W, Ws, scores0, Dk0, bias0, Dbias0):
    out = pl.pallas_call(
        _layer_in_body,
        grid=(NRB,),
        in_specs=[
            _bs((RB, 256), _row), _bs((256, 256), _full2), _bs((256, 256), _full2),
            _bs((256, 1), _full2), _bs((256, 1), _full2),
            _bs((1, 1), _full2), _bs((1, 1), _full2),
        ],
        out_specs=[
            _bs((2, RB, FH), _s3), _bs((RB, 256), _row),
            _bs((RB, 1), _row), _bs((RB, 1), _row),
        ],
        out_shape=[
            jax.ShapeDtypeStruct((2, N, FH), jnp.float32),
            jax.ShapeDtypeStruct((N, 256), jnp.float32),
            jax.ShapeDtypeStruct((N, 1), jnp.float32),
            jax.ShapeDtypeStruct((N, 1), jnp.float32),
        ],
    )(fea, W, Ws, scores0, Dk0, bias0.reshape(1, 1), Dbias0.reshape(1, 1))
    h2, hs, si, dki = out
    return h2.reshape(2 * N, FH), hs, si, dki


def _layer_mid_body(aa_ref, ab_ref, ka_ref, kb_ref, ha_ref, hb_ref, hs_ref,
                    si_ref, dki_ref, b_ref, wm_ref, wsm_ref, bm_ref,
                    ym2_ref, zm_ref):
    a = jnp.concatenate([aa_ref[...], ab_ref[...]], axis=1)
    k = jnp.concatenate([ka_ref[...], kb_ref[...]], axis=1)
    h = jnp.concatenate([ha_ref[...], hb_ref[...]], axis=1)
    hs = hs_ref[...]
    si = si_ref[...]
    dki = dki_ref[...]
    b = b_ref[...][None, :]
    x1 = si * a + (1.0 - si) * k + hs + b + GAMMA * dki * (h + hs + b)
    ym = jnp.dot(x1, wm_ref[...], preferred_element_type=jnp.float32)
    zm = jnp.dot(x1, wsm_ref[...], preferred_element_type=jnp.float32)
    ym2_ref[0] = ym[:, :FH]
    ym2_ref[1] = ym[:, FH:]
    zm_ref[...] = zm + bm_ref[...][None, :]


def _layer_mid(a2, k2, h2, hs, si, dki, b_in, W_mid, Ws_mid, b_mid):
    out = pl.pallas_call(
        _layer_mid_body,
        grid=(NRB,),
        in_specs=[
            _bs((RB, FH), _rowlo), _bs((RB, FH), _rowhi),
            _bs((RB, FH), _rowlo), _bs((RB, FH), _rowhi),
            _bs((RB, FH), _rowlo), _bs((RB, FH), _rowhi),
            _bs((RB, 256), _row), _bs((RB, 1), _row), _bs((RB, 1), _row),
            _bs((256,), _full1),
            _bs((256, 256), _full2), _bs((256, 256), _full2), _bs((256,), _full1),
        ],
        out_specs=[_bs((2, RB, FH), _s3), _bs((RB, 256), _row)],
        out_shape=[
            jax.ShapeDtypeStruct((2, N, FH), jnp.float32),
            jax.ShapeDtypeStruct((N, 256), jnp.float32),
        ],
    )(a2, a2, k2, k2, h2, h2, hs, si, dki, b_in, W_mid, Ws_mid, b_mid)
    ym2, zm = out
    return ym2.reshape(2 * N, FH), zm


def _layer_out_body(ma_ref, mb_ref, zm_ref, sc_ref, dk_ref, b1_ref, d1_ref,
                    wo_ref, wso_ref, g128_ref, gs_ref, so_ref, dko_ref):
    x2 = jnp.concatenate([ma_ref[...], mb_ref[...]], axis=1) + zm_ref[...]
    so_ref[...] = jax.nn.sigmoid(
        jnp.dot(x2, sc_ref[...], preferred_element_type=jnp.float32) + b1_ref[0, 0])
    dko_ref[...] = jnp.dot(x2, dk_ref[...],
                           preferred_element_type=jnp.float32) + d1_ref[0, 0]
    g = jnp.dot(x2, wo_ref[...], preferred_element_type=jnp.float32)
    gs_ref[...] = jnp.dot(x2, wso_ref[...], preferred_element_type=jnp.float32)
    g128_ref[...] = jnp.pad(g, ((0, 0), (0, FH - NCLASS)))


NCLASS = 64


def _layer_out(m2, zm, scores1, Dk1, bias1, Dbias1, W_out, Ws_out):
    return pl.pallas_call(
        _layer_out_body,
        grid=(NRB,),
        in_specs=[
            _bs((RB, FH), _rowlo), _bs((RB, FH), _rowhi),
            _bs((RB, 256), _row),
            _bs((256, 1), _full2), _bs((256, 1), _full2),
            _bs((1, 1), _full2), _bs((1, 1), _full2),
            _bs((256, NCLASS), _full2), _bs((256, NCLASS), _full2),
        ],
        out_specs=[
            _bs((RB, FH), _row), _bs((RB, NCLASS), _row),
            _bs((RB, 1), _row), _bs((RB, 1), _row),
        ],
        out_shape=[
            jax.ShapeDtypeStruct((N, FH), jnp.float32),
            jax.ShapeDtypeStruct((N, NCLASS), jnp.float32),
            jax.ShapeDtypeStruct((N, 1), jnp.float32),
            jax.ShapeDtypeStruct((N, 1), jnp.float32),
        ],
    )(m2, m2, zm, scores1, Dk1, bias1.reshape(1, 1), Dbias1.reshape(1, 1),
      W_out, Ws_out)


def _outcomb_body(oaa_ref, oab_ref, oka_ref, okb_ref, g128_ref, gs_ref,
                  s_ref, dk_ref, b_ref, o_ref):
    oa = oaa_ref[...][:, :NCLASS] + oab_ref[...][:, :NCLASS]
    ok = oka_ref[...][:, :NCLASS] + okb_ref[...][:, :NCLASS]
    g = g128_ref[...][:, :NCLASS]
    s = s_ref[...]
    dk = dk_ref[...]
    b = b_ref[...][None, :]
    gs = gs_ref[...]
    idt = g + gs + b
    x = s * oa + (1.0 - s) * ok + gs + b + GAMMA * dk * idt
    m = jnp.max(x, axis=1, keepdims=True)
    lse = jnp.log(jnp.sum(jnp.exp(x - m), axis=1, keepdims=True)) + m
    o_ref[...] = x - lse


def _outcomb(oa2, ok2, g128, gs, s, dk, b):
    return pl.pallas_call(
        _outcomb_body,
        grid=(NRB,),
        in_specs=[
            _bs((RB, FH), _rowlo), _bs((RB, FH), _rowhi),
            _bs((RB, FH), _rowlo), _bs((RB, FH), _rowhi),
            _bs((RB, FH), _row), _bs((RB, NCLASS), _row),
            _bs((RB, 1), _row), _bs((RB, 1), _row),
            _bs((NCLASS,), _full1),
        ],
        out_specs=_bs((RB, NCLASS), _row),
        out_shape=jax.ShapeDtypeStruct((N, NCLASS), jnp.float32),
    )(oa2, oa2, ok2, ok2, g128, gs, s, dk, b)


def kernel(fea, adj_edge_index, adj_values, adj_knn_edge_index, adj_knn_values,
           W_in, Ws_in, b_in, W_mid, Ws_mid, b_mid, W_out, Ws_out, b_out,
           scores0, bias0, scores1, bias1, Dk0, Dbias0, Dk1, Dbias1):
    ea = _prep_edges(adj_edge_index, adj_values)
    ek = _prep_edges(adj_knn_edge_index, adj_knn_values)
    zh = jnp.zeros((RPT, FH), jnp.float32)

    spmm_h = _get_spmm(False)
    spmm_o = _get_spmm(True)
    h2, hs, si, dki = _layer_in(fea, W_in, Ws_in, scores0, Dk0, bias0, Dbias0)
    a2 = spmm_h(h2, ea[0], ea[2], zh)
    k2 = spmm_h(h2, ek[0], ek[2], zh)
    ym2, zm = _layer_mid(a2, k2, h2, hs, si, dki, b_in, W_mid, Ws_mid, b_mid)
    m2 = spmm_h(ym2, ea[0], ea[2], zh)
    g128, gs, so, dko = _layer_out(m2, zm, scores1, Dk1, bias1, Dbias1,
                                   W_out, Ws_out)
    oa2 = spmm_o(g128, ea[1], ea[2], zh)
    ok2 = spmm_o(g128, ek[1], ek[2], zh)
    return _outcomb(oa2, ok2, g128, gs, so, dko, b_out)


# speedup vs baseline: 1.0182x; 1.0120x over previous
import functools

import jax
import jax.numpy as jnp
from jax import lax
from jax.experimental import pallas as pl
from jax.experimental.pallas import tpu as pltpu
from jax.experimental.pallas import tpu_sc as plsc

N = 10000
E = 160000
GAMMA = 0.1
RB = 400
NRB = N // RB

C = 128
NTILE = 16
FH = 128
E_PAD = 4096 * ((E + 4095) // 4096)
RPT = 624
TAIL = N - NTILE * RPT


def _make_spmm(edge_split, dual):
    per_tile = E_PAD // (32 if edge_split else 16)
    n_chunks = per_tile // C
    n_groups = n_chunks // 8
    n_passes = 2 if dual else 1
    out_sds = jax.ShapeDtypeStruct((2 * N, FH), jnp.float32)
    mesh = plsc.VectorSubcoreMesh(core_axis_name="c", subcore_axis_name="s",
                                  num_cores=2, num_subcores=NTILE)

    @functools.partial(
        pl.kernel,
        out_type=[out_sds] * n_passes,
        mesh=mesh,
        scratch_types=[
            pltpu.VMEM((8, 2, C), jnp.int32),
            pltpu.VMEM((4 * C * 16,), jnp.float32),
            pltpu.VMEM((2, C, FH), jnp.float32),
            pltpu.VMEM_SHARED((N, FH), jnp.float32),
        ] + [pltpu.SemaphoreType.DMA] * 16,
    )
    def spmm(table_hbm, *rest):
        edge_hbm = rest[:2 * n_passes]
        zeros_hbm = rest[2 * n_passes]
        out_hbms = rest[2 * n_passes + 1:3 * n_passes + 1]
        pk_v, vals_v, rows, acc = rest[3 * n_passes + 1:3 * n_passes + 5]
        sems = rest[3 * n_passes + 5:]
        psem = sems[0:8]
        vsem = sems[8:12]
        gsem = sems[12:14]
        ssem = sems[14:16]
        c = lax.axis_index("c")
        s = lax.axis_index("s")
        base = s * per_tile
        if edge_split:
            base = base + c * (E_PAD // 2)

        VCH = C * 16

        def run_pass(pk_hbm, vals_hbm, out_hbm, first):
            def issue_pk(j, bq):
                off = pl.multiple_of(base + j * C, C)
                pltpu.async_copy(pk_hbm.at[c, :, pl.ds(off, C)], pk_v.at[bq],
                                 psem[bq])
                bv = bq % 4
                pltpu.async_copy(vals_hbm.at[pl.ds(off * 16, VCH)],
                                 vals_v.at[pl.ds(bv * VCH, VCH)], vsem[bv])

            def drain_pk(bq):
                pltpu.make_async_copy(pk_hbm.at[c, :, pl.ds(0, C)],
                                      pk_v.at[bq], psem[bq]).wait()

            def drain_vals(bv):
                pltpu.make_async_copy(vals_hbm.at[pl.ds(0, VCH)],
                                      vals_v.at[pl.ds(bv * VCH, VCH)],
                                      vsem[bv]).wait()

            def issue_gather(bq, br):
                pltpu.async_copy(table_hbm.at[pk_v.at[bq, 0]], rows.at[br],
                                 gsem[br])

            def drain_gather(br):
                pltpu.make_async_copy(table_hbm.at[pl.ds(0, C)],
                                      rows.at[br], gsem[br]).wait()

            def issue_scatter(bq, br):
                pltpu.async_copy(rows.at[br], acc.at[pk_v.at[bq, 1]], ssem[br],
                                 add=True)

            def drain_scatter(br):
                pltpu.make_async_copy(table_hbm.at[pl.ds(0, C)],
                                      rows.at[br], ssem[br]).wait()

            for q in range(4):
                issue_pk(q, q)
            drain_pk(0)
            issue_gather(0, 0)

            pltpu.sync_copy(zeros_hbm, acc.at[pl.ds(s * RPT, RPT)])

            @pl.when(s == NTILE - 1)
            def _zero_tail():
                pltpu.sync_copy(zeros_hbm.at[pl.ds(0, TAIL)],
                                acc.at[pl.ds(NTILE * RPT, TAIL)])

            plsc.subcore_barrier()

            def group(gi, cy):
                for u in range(8):
                    br = u % 2
                    bv = u % 4
                    bq = u
                    j = 8 * gi + u
                    drain_gather(br)

                    @pl.when(j >= 1)
                    def _free(j=j, br=br):
                        drain_scatter(1 - br)

                    @pl.when(j + 1 < n_chunks)
                    def _prep(j=j, br=br, bq=bq):
                        drain_pk((bq + 1) % 8)
                        issue_gather((bq + 1) % 8, 1 - br)

                    drain_vals(bv)

                    def scale(i, cy2, br=br, bv=bv):
                        for e2 in range(2):
                            e = i * 2 + e2
                            vs = vals_v[pl.ds(bv * VCH + e * 16, 16)]
                            for f in range(FH // 16):
                                sl = pl.ds(f * 16, 16)
                                rows[br, e, sl] = rows[br, e, sl] * vs
                        return cy2

                    lax.fori_loop(0, C // 2, scale, 0)
                    issue_scatter(bq, br)

                    @pl.when(j + 4 < n_chunks)
                    def _next(j=j, bq=bq):
                        issue_pk(j + 4, (bq + 4) % 8)

                return cy

            lax.fori_loop(0, n_groups, group, 0)
            drain_scatter((n_chunks - 1) % 2)
            plsc.subcore_barrier()
            pltpu.sync_copy(acc.at[pl.ds(s * RPT, RPT)],
                            out_hbm.at[pl.ds(c * N + s * RPT, RPT)])

            @pl.when(s == NTILE - 1)
            def _flush_tail():
                pltpu.sync_copy(acc.at[pl.ds(NTILE * RPT, TAIL)],
                                out_hbm.at[pl.ds(c * N + NTILE * RPT, TAIL)])

        for p in range(n_passes):
            if p > 0:
                plsc.subcore_barrier()
            run_pass(edge_hbm[2 * p], edge_hbm[2 * p + 1], out_hbms[p],
                     p == 0)

    return spmm


@functools.cache
def _get_spmm(edge_split, dual):
    return _make_spmm(edge_split, dual)


def _prep_edges(ei, vals):
    pad = E_PAD - E
    src = jnp.concatenate([ei[1], jnp.zeros((pad,), jnp.int32)])
    dst = jnp.concatenate([ei[0], jnp.zeros((pad,), jnp.int32)])
    v = jnp.concatenate([vals, jnp.zeros((pad,), jnp.float32)])
    vx = jnp.broadcast_to(v[:, None], (E_PAD, 16)).reshape(-1)
    pkc = jnp.stack([jnp.stack([src, dst]), jnp.stack([src + N, dst])])
    pke = jnp.stack([jnp.stack([src, dst])] * 2)
    return pkc, pke, vx


_row = lambda i: (i, 0)
_rowlo = lambda i: (i, 0)
_rowhi = lambda i: (i + NRB, 0)
_full2 = lambda i: (0, 0)
_full1 = lambda i: (0,)
_s3 = lambda i: (0, i, 0)


def _bs(shape, imap):
    return pl.BlockSpec(shape, imap)


def _layer_in_body(f_ref, w_ref, ws_ref, sc_ref, dk_ref, b0_ref, d0_ref,
                   h2_ref, hs_ref, si_ref, dki_ref):
    f = f_ref[...]
    h = jnp.dot(f, w_ref[...], preferred_element_type=jnp.float32)
    hs = jnp.dot(f, ws_ref[...], preferred_element_type=jnp.float32)
    si_ref[...] = jax.nn.sigmoid(
        jnp.dot(f, sc_ref[...], preferred_element_type=jnp.float32) + b0_ref[0, 0])
    dki_ref[...] = jnp.dot(f, dk_ref[...],
                           preferred_element_type=jnp.float32) + d0_ref[0, 0]
    h2_ref[0] = h[:, :FH]
    h2_ref[1] = h[:, FH:]
    hs_ref[...] = hs


def _layer_in(fea, W, Ws, scores0, Dk0, bias0, Dbias0):
    out = pl.pallas_call(
        _layer_in_body,
        grid=(NRB,),
        in_specs=[
            _bs((RB, 256), _row), _bs((256, 256), _full2), _bs((256, 256), _full2),
            _bs((256, 1), _full2), _bs((256, 1), _full2),
            _bs((1, 1), _full2), _bs((1, 1), _full2),
        ],
        out_specs=[
            _bs((2, RB, FH), _s3), _bs((RB, 256), _row),
            _bs((RB, 1), _row), _bs((RB, 1), _row),
        ],
        out_shape=[
            jax.ShapeDtypeStruct((2, N, FH), jnp.float32),
            jax.ShapeDtypeStruct((N, 256), jnp.float32),
            jax.ShapeDtypeStruct((N, 1), jnp.float32),
            jax.ShapeDtypeStruct((N, 1), jnp.float32),
        ],
    )(fea, W, Ws, scores0, Dk0, bias0.reshape(1, 1), Dbias0.reshape(1, 1))
    h2, hs, si, dki = out
    return h2.reshape(2 * N, FH), hs, si, dki


def _layer_mid_body(aa_ref, ab_ref, ka_ref, kb_ref, ha_ref, hb_ref, hs_ref,
                    si_ref, dki_ref, b_ref, wm_ref, wsm_ref, bm_ref,
                    ym2_ref, zm_ref):
    a = jnp.concatenate([aa_ref[...], ab_ref[...]], axis=1)
    k = jnp.concatenate([ka_ref[...], kb_ref[...]], axis=1)
    h = jnp.concatenate([ha_ref[...], hb_ref[...]], axis=1)
    hs = hs_ref[...]
    si = si_ref[...]
    dki = dki_ref[...]
    b = b_ref[...][None, :]
    x1 = si * a + (1.0 - si) * k + hs + b + GAMMA * dki * (h + hs + b)
    ym = jnp.dot(x1, wm_ref[...], preferred_element_type=jnp.float32)
    zm = jnp.dot(x1, wsm_ref[...], preferred_element_type=jnp.float32)
    ym2_ref[0] = ym[:, :FH]
    ym2_ref[1] = ym[:, FH:]
    zm_ref[...] = zm + bm_ref[...][None, :]


def _layer_mid(a2, k2, h2, hs, si, dki, b_in, W_mid, Ws_mid, b_mid):
    out = pl.pallas_call(
        _layer_mid_body,
        grid=(NRB,),
        in_specs=[
            _bs((RB, FH), _rowlo), _bs((RB, FH), _rowhi),
            _bs((RB, FH), _rowlo), _bs((RB, FH), _rowhi),
            _bs((RB, FH), _rowlo), _bs((RB, FH), _rowhi),
            _bs((RB, 256), _row), _bs((RB, 1), _row), _bs((RB, 1), _row),
            _bs((256,), _full1),
            _bs((256, 256), _full2), _bs((256, 256), _full2), _bs((256,), _full1),
        ],
        out_specs=[_bs((2, RB, FH), _s3), _bs((RB, 256), _row)],
        out_shape=[
            jax.ShapeDtypeStruct((2, N, FH), jnp.float32),
            jax.ShapeDtypeStruct((N, 256), jnp.float32),
        ],
    )(a2, a2, k2, k2, h2, h2, hs, si, dki, b_in, W_mid, Ws_mid, b_mid)
    ym2, zm = out
    return ym2.reshape(2 * N, FH), zm


def _layer_out_body(ma_ref, mb_ref, zm_ref, sc_ref, dk_ref, b1_ref, d1_ref,
                    wo_ref, wso_ref, g128_ref, gs_ref, so_ref, dko_ref):
    x2 = jnp.concatenate([ma_ref[...], mb_ref[...]], axis=1) + zm_ref[...]
    so_ref[...] = jax.nn.sigmoid(
        jnp.dot(x2, sc_ref[...], preferred_element_type=jnp.float32) + b1_ref[0, 0])
    dko_ref[...] = jnp.dot(x2, dk_ref[...],
                           preferred_element_type=jnp.float32) + d1_ref[0, 0]
    g = jnp.dot(x2, wo_ref[...], preferred_element_type=jnp.float32)
    gs_ref[...] = jnp.dot(x2, wso_ref[...], preferred_element_type=jnp.float32)
    g128_ref[...] = jnp.pad(g, ((0, 0), (0, FH - NCLASS)))


NCLASS = 64


def _layer_out(m2, zm, scores1, Dk1, bias1, Dbias1, W_out, Ws_out):
    return pl.pallas_call(
        _layer_out_body,
        grid=(NRB,),
        in_specs=[
            _bs((RB, FH), _rowlo), _bs((RB, FH), _rowhi),
            _bs((RB, 256), _row),
            _bs((256, 1), _full2), _bs((256, 1), _full2),
            _bs((1, 1), _full2), _bs((1, 1), _full2),
            _bs((256, NCLASS), _full2), _bs((256, NCLASS), _full2),
        ],
        out_specs=[
            _bs((RB, FH), _row), _bs((RB, NCLASS), _row),
            _bs((RB, 1), _row), _bs((RB, 1), _row),
        ],
        out_shape=[
            jax.ShapeDtypeStruct((N, FH), jnp.float32),
            jax.ShapeDtypeStruct((N, NCLASS), jnp.float32),
            jax.ShapeDtypeStruct((N, 1), jnp.float32),
            jax.ShapeDtypeStruct((N, 1), jnp.float32),
        ],
    )(m2, m2, zm, scores1, Dk1, bias1.reshape(1, 1), Dbias1.reshape(1, 1),
      W_out, Ws_out)


def _outcomb_body(oaa_ref, oab_ref, oka_ref, okb_ref, g128_ref, gs_ref,
                  s_ref, dk_ref, b_ref, o_ref):
    oa = oaa_ref[...][:, :NCLASS] + oab_ref[...][:, :NCLASS]
    ok = oka_ref[...][:, :NCLASS] + okb_ref[...][:, :NCLASS]
    g = g128_ref[...][:, :NCLASS]
    s = s_ref[...]
    dk = dk_ref[...]
    b = b_ref[...][None, :]
    gs = gs_ref[...]
    idt = g + gs + b
    x = s * oa + (1.0 - s) * ok + gs + b + GAMMA * dk * idt
    m = jnp.max(x, axis=1, keepdims=True)
    lse = jnp.log(jnp.sum(jnp.exp(x - m), axis=1, keepdims=True)) + m
    o_ref[...] = x - lse


def _outcomb(oa2, ok2, g128, gs, s, dk, b):
    return pl.pallas_call(
        _outcomb_body,
        grid=(NRB,),
        in_specs=[
            _bs((RB, FH), _rowlo), _bs((RB, FH), _rowhi),
            _bs((RB, FH), _rowlo), _bs((RB, FH), _rowhi),
            _bs((RB, FH), _row), _bs((RB, NCLASS), _row),
            _bs((RB, 1), _row), _bs((RB, 1), _row),
            _bs((NCLASS,), _full1),
        ],
        out_specs=_bs((RB, NCLASS), _row),
        out_shape=jax.ShapeDtypeStruct((N, NCLASS), jnp.float32),
    )(oa2, oa2, ok2, ok2, g128, gs, s, dk, b)


def kernel(fea, adj_edge_index, adj_values, adj_knn_edge_index, adj_knn_values,
           W_in, Ws_in, b_in, W_mid, Ws_mid, b_mid, W_out, Ws_out, b_out,
           scores0, bias0, scores1, bias1, Dk0, Dbias0, Dk1, Dbias1):
    ea = _prep_edges(adj_edge_index, adj_values)
    ek = _prep_edges(adj_knn_edge_index, adj_knn_values)
    zh = jnp.zeros((RPT, FH), jnp.float32)

    h2, hs, si, dki = _layer_in(fea, W_in, Ws_in, scores0, Dk0, bias0, Dbias0)
    a2, k2 = _get_spmm(False, True)(h2, ea[0], ea[2], ek[0], ek[2], zh)
    ym2, zm = _layer_mid(a2, k2, h2, hs, si, dki, b_in, W_mid, Ws_mid, b_mid)
    (m2,) = _get_spmm(False, False)(ym2, ea[0], ea[2], zh)
    g128, gs, so, dko = _layer_out(m2, zm, scores1, Dk1, bias1, Dbias1,
                                   W_out, Ws_out)
    oa2, ok2 = _get_spmm(True, True)(g128, ea[1], ea[2], ek[1], ek[2], zh)
    return _outcomb(oa2, ok2, g128, gs, so, dko, b_out)
